# baseline (device time: 33733 ns/iter reference)
import jax
import jax.numpy as jnp
from jax import lax
from jax.experimental import pallas as pl
from jax.experimental.pallas import tpu as pltpu

Z = 4
V_PER = 4096
T = 512
D = 512


def kernel(ids, E):
    def body(ids_ref, e_ref, out_ref, comm_ref, send_sems, recv_sems):
        my_x = lax.axis_index("x")
        my_y = lax.axis_index("y")
        my_z = lax.axis_index("z")
        left = (my_z - 1) % Z
        right = (my_z + 1) % Z

        barrier_sem = pltpu.get_barrier_semaphore()
        for nbr in (left, right):
            pl.semaphore_signal(
                barrier_sem, inc=1,
                device_id=(my_x, my_y, nbr),
                device_id_type=pl.DeviceIdType.MESH,
            )
        pl.semaphore_wait(barrier_sem, 2)

        local = ids_ref[:] - my_z * V_PER
        col = lax.broadcasted_iota(jnp.int32, (T, V_PER), 1)
        onehot = (col == local).astype(jnp.bfloat16)
        partial = jnp.dot(
            onehot, e_ref[:].astype(jnp.bfloat16),
            preferred_element_type=jnp.float32,
        )
        out_ref[:, :] = partial
        comm_ref[0, :, :] = partial.astype(jnp.bfloat16)

        for h in range(Z - 1):
            rdma = pltpu.make_async_remote_copy(
                src_ref=comm_ref.at[h],
                dst_ref=comm_ref.at[h + 1],
                send_sem=send_sems.at[h],
                recv_sem=recv_sems.at[h],
                device_id=(my_x, my_y, right),
                device_id_type=pl.DeviceIdType.MESH,
            )
            rdma.start()
            rdma.wait()
            out_ref[:, :] += comm_ref[h + 1, :, :].astype(jnp.float32)

    return pl.pallas_call(
        body,
        out_shape=jax.ShapeDtypeStruct((T, D), jnp.float32),
        in_specs=[
            pl.BlockSpec(memory_space=pltpu.VMEM),
            pl.BlockSpec(memory_space=pltpu.VMEM),
        ],
        out_specs=pl.BlockSpec(memory_space=pltpu.VMEM),
        scratch_shapes=[
            pltpu.VMEM((Z, T, D), jnp.bfloat16),
            pltpu.SemaphoreType.DMA((Z - 1,)),
            pltpu.SemaphoreType.DMA((Z - 1,)),
        ],
        compiler_params=pltpu.CompilerParams(collective_id=0),
    )(ids.reshape(T, 1), E)


# device time: 22026 ns/iter; 1.5315x vs baseline; 1.5315x over previous
import jax
import jax.numpy as jnp
from jax import lax
from jax.experimental import pallas as pl
from jax.experimental.pallas import tpu as pltpu

Z = 4
V_PER = 4096
T = 512
D = 512
T_CH = T // 4


def kernel(ids, E):
    def body(ids_ref, e_ref, out_ref,
             pbuf, zbuf, rbuf, xybuf,
             zsend, zrecv, xysend, xyrecv):
        my_x = lax.axis_index("x")
        my_y = lax.axis_index("y")
        my_z = lax.axis_index("z")
        q = 2 * my_x + my_y

        xy_peers = ((1 - my_x, my_y), (my_x, 1 - my_y), (1 - my_x, 1 - my_y))

        barrier_sem = pltpu.get_barrier_semaphore()
        for d in range(1, Z):
            pl.semaphore_signal(
                barrier_sem, inc=1,
                device_id=(my_x, my_y, (my_z + d) % Z),
                device_id_type=pl.DeviceIdType.MESH,
            )
        for px, py in xy_peers:
            pl.semaphore_signal(
                barrier_sem, inc=1,
                device_id=(px, py, my_z),
                device_id_type=pl.DeviceIdType.MESH,
            )
        pl.semaphore_wait(barrier_sem, 6)

        idc = ids_ref[pl.ds(q * T_CH, T_CH), :]
        local = idc - my_z * V_PER
        col = lax.broadcasted_iota(jnp.int32, (T_CH, V_PER), 1)
        onehot = (col == local).astype(jnp.bfloat16)
        partial = jnp.dot(
            onehot, e_ref[:].astype(jnp.bfloat16),
            preferred_element_type=jnp.float32,
        )
        pbuf[:, :] = partial.astype(jnp.bfloat16)

        zrdmas = []
        for d in range(1, Z):
            r = pltpu.make_async_remote_copy(
                src_ref=pbuf,
                dst_ref=zbuf.at[d - 1],
                send_sem=zsend.at[d - 1],
                recv_sem=zrecv.at[d - 1],
                device_id=(my_x, my_y, (my_z + d) % Z),
                device_id_type=pl.DeviceIdType.MESH,
            )
            r.start()
            zrdmas.append(r)

        red = partial
        for d in range(1, Z):
            zrdmas[d - 1].wait_recv()
            red = red + zbuf[d - 1, :, :].astype(jnp.float32)
        rbuf[:, :] = red.astype(jnp.bfloat16)
        out_ref[pl.ds(q * T_CH, T_CH), :] = red

        xyrdmas = []
        for k, (px, py) in enumerate(xy_peers):
            r = pltpu.make_async_remote_copy(
                src_ref=rbuf,
                dst_ref=xybuf.at[k],
                send_sem=xysend.at[k],
                recv_sem=xyrecv.at[k],
                device_id=(px, py, my_z),
                device_id_type=pl.DeviceIdType.MESH,
            )
            r.start()
            xyrdmas.append(r)

        for k, (px, py) in enumerate(xy_peers):
            qk = 2 * px + py
            xyrdmas[k].wait_recv()
            out_ref[pl.ds(qk * T_CH, T_CH), :] = (
                xybuf[k, :, :].astype(jnp.float32)
            )

        for r in zrdmas + xyrdmas:
            r.wait_send()

    return pl.pallas_call(
        body,
        out_shape=jax.ShapeDtypeStruct((T, D), jnp.float32),
        in_specs=[
            pl.BlockSpec(memory_space=pltpu.VMEM),
            pl.BlockSpec(memory_space=pltpu.VMEM),
        ],
        out_specs=pl.BlockSpec(memory_space=pltpu.VMEM),
        scratch_shapes=[
            pltpu.VMEM((T_CH, D), jnp.bfloat16),
            pltpu.VMEM((Z - 1, T_CH, D), jnp.bfloat16),
            pltpu.VMEM((T_CH, D), jnp.bfloat16),
            pltpu.VMEM((3, T_CH, D), jnp.bfloat16),
            pltpu.SemaphoreType.DMA((Z - 1,)),
            pltpu.SemaphoreType.DMA((Z - 1,)),
            pltpu.SemaphoreType.DMA((3,)),
            pltpu.SemaphoreType.DMA((3,)),
        ],
        compiler_params=pltpu.CompilerParams(collective_id=0),
    )(ids.reshape(T, 1), E)
